# baseline (speedup 1.0000x reference)
import functools
import math

import jax
import jax.numpy as jnp
import numpy as np
from jax.experimental import pallas as pl
from jax.experimental.pallas import tpu as pltpu

H = 96
QK = 8
VB = 8
W = 8
HALO = 16
T = 1024

_INV_SQRT_QK = 1.0 / math.sqrt(float(QK))
_STD_OF_BM = np.arange(H * QK).reshape(H, QK).T.reshape(-1)
_G_BM = np.tile(np.eye(H, dtype=np.float32), (QK, 1))

_NT = (((1,), (1,)), ((), ()))


def _rosa_body(h_ref, wq_ref, wk_ref, wv_ref, wo_ref, bias_ref, g_ref,
               demb_ref, out_ref, kext_ref, vext_ref):
    i = pl.program_id(0)
    h = h_ref[...].astype(jnp.bfloat16)
    q = jax.lax.dot_general(h, wq_ref[...], _NT,
                            preferred_element_type=jnp.float32)
    k = jax.lax.dot_general(h, wk_ref[...], _NT,
                            preferred_element_type=jnp.float32)
    v = jax.lax.dot_general(h, wv_ref[...], _NT,
                            preferred_element_type=jnp.float32)
    qb = jnp.tanh(q).astype(jnp.bfloat16)
    kf = jnp.tanh(k)
    vf = jax.nn.sigmoid(v) * demb_ref[...]
    kb = kf.astype(jnp.bfloat16)
    vb = vf.astype(jnp.bfloat16)

    @pl.when(i == 0)
    def _init_halo():
        kext_ref[0:HALO] = jnp.zeros((HALO, H * QK), jnp.float32)
        vext_ref[0:HALO] = jnp.zeros((HALO, H * VB), jnp.float32)

    @pl.when(i > 0)
    def _carry_halo():
        kext_ref[0:HALO] = kext_ref[T:T + HALO]
        vext_ref[0:HALO] = vext_ref[T:T + HALO]

    kext_ref[HALO:] = kf
    vext_ref[HALO:] = vf

    row = i * T + jax.lax.broadcasted_iota(jnp.int32, (T, H), 0)
    scores = []
    for o in range(W):
        prod = qb * (kb if o == 0 else
                     kext_ref[HALO - o:HALO - o + T].astype(jnp.bfloat16))
        s = jnp.dot(prod, g_ref[...],
                    preferred_element_type=jnp.float32) * _INV_SQRT_QK
        if o > 0:
            s = jnp.where(row >= o, s, -1e30)
        scores.append(s)
    exps = [jnp.exp(s) for s in scores]
    inv = 1.0 / functools.reduce(lambda a, b: a + b, exps)
    acc = None
    for o in range(W):
        p = (exps[o] * inv).astype(jnp.bfloat16)
        pw = jnp.concatenate([p] * VB, axis=1)
        term = pw * (vb if o == 0 else
                     vext_ref[HALO - o:HALO - o + T].astype(jnp.bfloat16))
        acc = term if acc is None else acc + term
    res = jax.lax.dot_general(acc, wo_ref[...], _NT,
                              preferred_element_type=jnp.float32)
    out_ref[...] = res + bias_ref[...]


def kernel(hidden_states, Wq, Wk, Wv, Wo, v_emb0, v_emb1):
    b, s, hid = hidden_states.shape
    h2 = hidden_states.reshape(b * s, hid)
    perm = _STD_OF_BM

    def _perm_rows(w):
        wb = w.astype(jnp.bfloat16)
        return wb.reshape(H, QK, hid).swapaxes(0, 1).reshape(H * QK, hid)

    wq = _perm_rows(Wq)
    wk = _perm_rows(Wk)
    wv = _perm_rows(Wv)
    wo = (Wo.astype(jnp.bfloat16)
          .reshape(hid, H, VB).swapaxes(1, 2).reshape(hid, H * VB))
    demb = (v_emb1 - v_emb0)[perm].reshape(1, H * VB)
    bias = (Wo @ v_emb0).reshape(1, hid)
    g = jnp.asarray(_G_BM, dtype=jnp.bfloat16)
    nb = (b * s) // T

    out = pl.pallas_call(
        _rosa_body,
        grid=(nb,),
        in_specs=[
            pl.BlockSpec((T, hid), lambda i: (i, 0)),
            pl.BlockSpec((H * QK, hid), lambda i: (0, 0)),
            pl.BlockSpec((H * QK, hid), lambda i: (0, 0)),
            pl.BlockSpec((H * VB, hid), lambda i: (0, 0)),
            pl.BlockSpec((hid, H * VB), lambda i: (0, 0)),
            pl.BlockSpec((1, hid), lambda i: (0, 0)),
            pl.BlockSpec((H * QK, H), lambda i: (0, 0)),
            pl.BlockSpec((1, H * VB), lambda i: (0, 0)),
        ],
        out_specs=pl.BlockSpec((T, hid), lambda i: (i, 0)),
        out_shape=jax.ShapeDtypeStruct((b * s, hid), jnp.float32),
        scratch_shapes=[
            pltpu.VMEM((T + HALO, H * QK), jnp.float32),
            pltpu.VMEM((T + HALO, H * VB), jnp.float32),
        ],
    )(h2, wq, wk, wv, wo, bias, g, demb)
    return out.reshape(b, s, hid)


# speedup vs baseline: 1.1664x; 1.1664x over previous
import functools
import math

import jax
import jax.numpy as jnp
import numpy as np
from jax.experimental import pallas as pl
from jax.experimental.pallas import tpu as pltpu

H = 96
QK = 8
VB = 8
W = 8
HALO = 16
T = 1024

_INV_SQRT_QK = 1.0 / math.sqrt(float(QK))
_STD_OF_BM = np.arange(H * QK).reshape(H, QK).T.reshape(-1)
_P_BM = np.zeros((H * QK, H * QK), dtype=np.float32)
_P_BM[np.arange(H * QK), _STD_OF_BM] = 1.0
_G_BM = np.tile(np.eye(H, dtype=np.float32), (QK, 1))

_NT = (((1,), (1,)), ((), ()))


def _rosa_body(h_ref, wq_ref, wk_ref, wv_ref, wo_ref, ve0_ref, ve1_ref,
               p_ref, g_ref, out_ref,
               wqp_ref, wkp_ref, wvp_ref, wop_ref, demb_ref, bias_ref,
               kext_ref, vext_ref):
    i = pl.program_id(0)

    @pl.when(i == 0)
    def _prep():
        pm = p_ref[...]
        wqp_ref[...] = jnp.dot(
            pm, wq_ref[...].astype(jnp.bfloat16),
            preferred_element_type=jnp.float32).astype(jnp.bfloat16)
        wkp_ref[...] = jnp.dot(
            pm, wk_ref[...].astype(jnp.bfloat16),
            preferred_element_type=jnp.float32).astype(jnp.bfloat16)
        wvp_ref[...] = jnp.dot(
            pm, wv_ref[...].astype(jnp.bfloat16),
            preferred_element_type=jnp.float32).astype(jnp.bfloat16)
        wo_bf = wo_ref[...].astype(jnp.bfloat16)
        wop_ref[...] = jax.lax.dot_general(
            wo_bf, pm, _NT,
            preferred_element_type=jnp.float32).astype(jnp.bfloat16)
        demb_std = (ve1_ref[...] - ve0_ref[...]).astype(jnp.bfloat16)
        demb_ref[...] = jax.lax.dot_general(
            demb_std, pm, _NT, preferred_element_type=jnp.float32)
        bias_ref[...] = jax.lax.dot_general(
            ve0_ref[...].astype(jnp.bfloat16), wo_bf, _NT,
            preferred_element_type=jnp.float32)
        kext_ref[0:HALO] = jnp.zeros((HALO, H * QK), jnp.float32)
        vext_ref[0:HALO] = jnp.zeros((HALO, H * VB), jnp.float32)

    @pl.when(i > 0)
    def _carry_halo():
        kext_ref[0:HALO] = kext_ref[T:T + HALO]
        vext_ref[0:HALO] = vext_ref[T:T + HALO]

    h = h_ref[...].astype(jnp.bfloat16)
    q = jax.lax.dot_general(h, wqp_ref[...], _NT,
                            preferred_element_type=jnp.float32)
    k = jax.lax.dot_general(h, wkp_ref[...], _NT,
                            preferred_element_type=jnp.float32)
    v = jax.lax.dot_general(h, wvp_ref[...], _NT,
                            preferred_element_type=jnp.float32)
    qb = jnp.tanh(q).astype(jnp.bfloat16)
    kf = jnp.tanh(k)
    vf = jax.nn.sigmoid(v) * demb_ref[...]
    kb = kf.astype(jnp.bfloat16)
    vb = vf.astype(jnp.bfloat16)

    kext_ref[HALO:] = kf
    vext_ref[HALO:] = vf

    row = i * T + jax.lax.broadcasted_iota(jnp.int32, (T, H), 0)
    scores = []
    for o in range(W):
        prod = qb * (kb if o == 0 else
                     kext_ref[HALO - o:HALO - o + T].astype(jnp.bfloat16))
        s = jnp.dot(prod, g_ref[...],
                    preferred_element_type=jnp.float32) * _INV_SQRT_QK
        if o > 0:
            s = jnp.where(row >= o, s, -1e30)
        scores.append(s)
    exps = [jnp.exp(s) for s in scores]
    inv = 1.0 / functools.reduce(lambda a, b: a + b, exps)
    acc = None
    for o in range(W):
        p = (exps[o] * inv).astype(jnp.bfloat16)
        pw = jnp.concatenate([p] * VB, axis=1)
        term = pw * (vb if o == 0 else
                     vext_ref[HALO - o:HALO - o + T].astype(jnp.bfloat16))
        acc = term if acc is None else acc + term
    res = jax.lax.dot_general(acc, wop_ref[...], _NT,
                              preferred_element_type=jnp.float32)
    out_ref[...] = res + bias_ref[...]


def kernel(hidden_states, Wq, Wk, Wv, Wo, v_emb0, v_emb1):
    b, s, hid = hidden_states.shape
    h2 = hidden_states.reshape(b * s, hid)
    p = jnp.asarray(_P_BM, dtype=jnp.bfloat16)
    g = jnp.asarray(_G_BM, dtype=jnp.bfloat16)
    nb = (b * s) // T

    out = pl.pallas_call(
        _rosa_body,
        grid=(nb,),
        in_specs=[
            pl.BlockSpec((T, hid), lambda i: (i, 0)),
            pl.BlockSpec((H * QK, hid), lambda i: (0, 0)),
            pl.BlockSpec((H * QK, hid), lambda i: (0, 0)),
            pl.BlockSpec((H * VB, hid), lambda i: (0, 0)),
            pl.BlockSpec((hid, H * VB), lambda i: (0, 0)),
            pl.BlockSpec((1, hid), lambda i: (0, 0)),
            pl.BlockSpec((1, hid), lambda i: (0, 0)),
            pl.BlockSpec((H * QK, H * QK), lambda i: (0, 0)),
            pl.BlockSpec((H * QK, H), lambda i: (0, 0)),
        ],
        out_specs=pl.BlockSpec((T, hid), lambda i: (i, 0)),
        out_shape=jax.ShapeDtypeStruct((b * s, hid), jnp.float32),
        scratch_shapes=[
            pltpu.VMEM((H * QK, hid), jnp.bfloat16),
            pltpu.VMEM((H * QK, hid), jnp.bfloat16),
            pltpu.VMEM((H * VB, hid), jnp.bfloat16),
            pltpu.VMEM((hid, H * VB), jnp.bfloat16),
            pltpu.VMEM((1, H * VB), jnp.float32),
            pltpu.VMEM((1, hid), jnp.float32),
            pltpu.VMEM((T + HALO, H * QK), jnp.float32),
            pltpu.VMEM((T + HALO, H * VB), jnp.float32),
        ],
    )(h2, Wq, Wk, Wv, Wo, v_emb0.reshape(1, hid), v_emb1.reshape(1, hid), p, g)
    return out.reshape(b, s, hid)


# speedup vs baseline: 1.2010x; 1.0296x over previous
import functools
import math

import jax
import jax.numpy as jnp
import numpy as np
from jax.experimental import pallas as pl
from jax.experimental.pallas import tpu as pltpu

H = 96
QK = 8
VB = 8
W = 8
HALO = 16
T = 1024

_INV_SQRT_QK = 1.0 / math.sqrt(float(QK))
_STD_OF_BM = np.arange(H * QK).reshape(H, QK).T.reshape(-1)
_P_BM = np.zeros((H * QK, H * QK), dtype=np.float32)
_P_BM[np.arange(H * QK), _STD_OF_BM] = 1.0
_G_BM = np.tile(np.eye(H, dtype=np.float32), (QK, 1))

_NT = (((1,), (1,)), ((), ()))


def _rosa_body(h_ref, wq_ref, wk_ref, wv_ref, wo_ref, ve0_ref, ve1_ref,
               p_ref, g_ref, out_ref,
               wqp_ref, wkp_ref, wvp_ref, wop_ref, demb_ref, bias_ref,
               kext_ref, vext_ref):
    i = pl.program_id(0)

    @pl.when(i == 0)
    def _prep():
        pm = p_ref[...]
        wqp_ref[...] = jnp.dot(
            pm, wq_ref[...].astype(jnp.bfloat16),
            preferred_element_type=jnp.float32).astype(jnp.bfloat16)
        wkp_ref[...] = jnp.dot(
            pm, wk_ref[...].astype(jnp.bfloat16),
            preferred_element_type=jnp.float32).astype(jnp.bfloat16)
        wvp_ref[...] = jnp.dot(
            pm, wv_ref[...].astype(jnp.bfloat16),
            preferred_element_type=jnp.float32).astype(jnp.bfloat16)
        wo_bf = wo_ref[...].astype(jnp.bfloat16)
        wop_ref[...] = jax.lax.dot_general(
            wo_bf, pm, _NT,
            preferred_element_type=jnp.float32).astype(jnp.bfloat16)
        demb_std = (ve1_ref[...] - ve0_ref[...]).astype(jnp.bfloat16)
        demb_ref[...] = jax.lax.dot_general(
            demb_std, pm, _NT, preferred_element_type=jnp.float32)
        bias_ref[...] = jax.lax.dot_general(
            ve0_ref[...].astype(jnp.bfloat16), wo_bf, _NT,
            preferred_element_type=jnp.float32)
        kext_ref[0:HALO] = jnp.zeros((HALO, H * QK), jnp.float32)
        vext_ref[0:HALO] = jnp.zeros((HALO, H * VB), jnp.float32)

    @pl.when(i > 0)
    def _carry_halo():
        kext_ref[0:HALO] = kext_ref[T:T + HALO]
        vext_ref[0:HALO] = vext_ref[T:T + HALO]

    h = h_ref[...].astype(jnp.bfloat16)
    q = jax.lax.dot_general(h, wqp_ref[...], _NT,
                            preferred_element_type=jnp.float32)
    k = jax.lax.dot_general(h, wkp_ref[...], _NT,
                            preferred_element_type=jnp.float32)
    v = jax.lax.dot_general(h, wvp_ref[...], _NT,
                            preferred_element_type=jnp.float32)
    qb = jnp.tanh(q).astype(jnp.bfloat16)
    kf = jnp.tanh(k)
    vf = jax.nn.sigmoid(v) * demb_ref[...]
    kb = kf.astype(jnp.bfloat16)
    vb = vf.astype(jnp.bfloat16)

    kext_ref[HALO:] = kf
    vext_ref[HALO:] = vf

    row = i * T + jax.lax.broadcasted_iota(jnp.int32, (T, H), 0)
    scores = []
    for o in range(W):
        prod = qb * (kb if o == 0 else
                     kext_ref[HALO - o:HALO - o + T].astype(jnp.bfloat16))
        s = jnp.dot(prod, g_ref[...],
                    preferred_element_type=jnp.float32) * _INV_SQRT_QK
        if o > 0:
            s = jnp.where(row >= o, s, -1e30)
        scores.append(s)
    exps = [jnp.exp(s) for s in scores]
    acc = None
    for o in range(W):
        ew = jnp.concatenate([exps[o].astype(jnp.bfloat16)] * VB,
                             axis=1)
        term = ew * (vb if o == 0 else
                     vext_ref[HALO - o:HALO - o + T].astype(jnp.bfloat16))
        acc = term if acc is None else acc + term
    inv = 1.0 / functools.reduce(lambda a, b: a + b, exps)
    invw = jnp.concatenate([inv.astype(jnp.bfloat16)] * VB, axis=1)
    res = jax.lax.dot_general(acc * invw, wop_ref[...], _NT,
                              preferred_element_type=jnp.float32)
    out_ref[...] = res + bias_ref[...]


def kernel(hidden_states, Wq, Wk, Wv, Wo, v_emb0, v_emb1):
    b, s, hid = hidden_states.shape
    h2 = hidden_states.reshape(b * s, hid)
    p = jnp.asarray(_P_BM, dtype=jnp.bfloat16)
    g = jnp.asarray(_G_BM, dtype=jnp.bfloat16)
    nb = (b * s) // T

    out = pl.pallas_call(
        _rosa_body,
        grid=(nb,),
        in_specs=[
            pl.BlockSpec((T, hid), lambda i: (i, 0)),
            pl.BlockSpec((H * QK, hid), lambda i: (0, 0)),
            pl.BlockSpec((H * QK, hid), lambda i: (0, 0)),
            pl.BlockSpec((H * VB, hid), lambda i: (0, 0)),
            pl.BlockSpec((hid, H * VB), lambda i: (0, 0)),
            pl.BlockSpec((1, hid), lambda i: (0, 0)),
            pl.BlockSpec((1, hid), lambda i: (0, 0)),
            pl.BlockSpec((H * QK, H * QK), lambda i: (0, 0)),
            pl.BlockSpec((H * QK, H), lambda i: (0, 0)),
        ],
        out_specs=pl.BlockSpec((T, hid), lambda i: (i, 0)),
        out_shape=jax.ShapeDtypeStruct((b * s, hid), jnp.float32),
        scratch_shapes=[
            pltpu.VMEM((H * QK, hid), jnp.bfloat16),
            pltpu.VMEM((H * QK, hid), jnp.bfloat16),
            pltpu.VMEM((H * VB, hid), jnp.bfloat16),
            pltpu.VMEM((hid, H * VB), jnp.bfloat16),
            pltpu.VMEM((1, H * VB), jnp.float32),
            pltpu.VMEM((1, hid), jnp.float32),
            pltpu.VMEM((T + HALO, H * QK), jnp.float32),
            pltpu.VMEM((T + HALO, H * VB), jnp.float32),
        ],
    )(h2, Wq, Wk, Wv, Wo, v_emb0.reshape(1, hid), v_emb1.reshape(1, hid), p, g)
    return out.reshape(b, s, hid)
